# Initial kernel scaffold; baseline (speedup 1.0000x reference)
import jax
import jax.numpy as jnp
from jax.experimental import pallas as pl


def kernel(x, router_w, gate_up_w, down_w):
    raise NotImplementedError("write your pallas kernel here")


# speedup vs baseline: 1.7657x; 1.7657x over previous
import functools

import jax
import jax.numpy as jnp
from jax.experimental import pallas as pl
from jax.experimental.pallas import tpu as pltpu

N_TOK = 2048
D_MODEL = 768
D_FF = 3072
N_EXP = 8
FC = 768
NF = D_FF // FC


def _ffn_body(x_ref, rw_ref, g_ref, u_ref, d_ref, o_ref, comb_ref):
    e = pl.program_id(0)
    f = pl.program_id(1)
    is_first = jnp.logical_and(e == 0, f == 0)

    @pl.when(is_first)
    def _():
        x = x_ref[...]
        logits = jax.lax.dot_general(
            x, rw_ref[...], (((1,), (1,)), ((), ())),
            preferred_element_type=jnp.float32,
        )
        m = jnp.max(logits, axis=1, keepdims=True)
        z = jnp.exp(logits - m)
        p = z / jnp.sum(z, axis=1, keepdims=True)
        lane = jax.lax.broadcasted_iota(jnp.int32, (N_TOK, N_EXP), 1)
        m1 = jnp.max(p, axis=1, keepdims=True)
        e1 = jnp.min(jnp.where(p == m1, lane, N_EXP), axis=1, keepdims=True)
        mask1 = lane == e1
        p2 = jnp.where(mask1, -1.0, p)
        m2 = jnp.max(p2, axis=1, keepdims=True)
        e2 = jnp.min(jnp.where(p2 == m2, lane, N_EXP), axis=1, keepdims=True)
        mask2 = lane == e2
        denom = m1 + m2 + 1e-8
        comb_ref[...] = jnp.where(mask1, m1, jnp.where(mask2, m2, 0.0)) / denom

    x = x_ref[...]
    lane = jax.lax.broadcasted_iota(jnp.int32, (N_TOK, N_EXP), 1)
    s = jnp.sum(jnp.where(lane == e, comb_ref[...], 0.0), axis=1, keepdims=True)

    gw = g_ref[0, 0]
    uw = u_ref[0, 0]
    dw = d_ref[0]
    g = jax.lax.dot_general(x, gw, (((1,), (1,)), ((), ())),
                            preferred_element_type=jnp.float32)
    u = jax.lax.dot_general(x, uw, (((1,), (1,)), ((), ())),
                            preferred_element_type=jnp.float32)
    act = g * jax.nn.sigmoid(g) * u
    y = jax.lax.dot_general(act, dw, (((1,), (1,)), ((), ())),
                            preferred_element_type=jnp.float32)
    contrib = y * s

    @pl.when(is_first)
    def _():
        o_ref[...] = contrib

    @pl.when(jnp.logical_not(is_first))
    def _():
        o_ref[...] = o_ref[...] + contrib


@jax.jit
def kernel(x, router_w, gate_up_w, down_w):
    gu = gate_up_w.reshape(N_EXP, 2, D_FF, D_MODEL)
    out = pl.pallas_call(
        _ffn_body,
        grid=(N_EXP, NF),
        in_specs=[
            pl.BlockSpec((N_TOK, D_MODEL), lambda e, f: (0, 0)),
            pl.BlockSpec((N_EXP, D_MODEL), lambda e, f: (0, 0)),
            pl.BlockSpec((1, 1, FC, D_MODEL), lambda e, f: (e, 0, f, 0)),
            pl.BlockSpec((1, 1, FC, D_MODEL), lambda e, f: (e, 1, f, 0)),
            pl.BlockSpec((1, D_MODEL, FC), lambda e, f: (e, 0, f)),
        ],
        out_specs=pl.BlockSpec((N_TOK, D_MODEL), lambda e, f: (0, 0)),
        out_shape=jax.ShapeDtypeStruct((N_TOK, D_MODEL), jnp.float32),
        scratch_shapes=[pltpu.VMEM((N_TOK, N_EXP), jnp.float32)],
        compiler_params=pltpu.CompilerParams(
            dimension_semantics=("arbitrary", "arbitrary"),
        ),
    )(x, router_w, gu, gu, down_w)
    return out
